# baseline (speedup 1.0000x reference)
import functools

import jax
import jax.numpy as jnp
from jax import lax
from jax.experimental import pallas as pl
from jax.experimental.pallas import tpu as pltpu
from jax.experimental.pallas import tpu_sc as plsc

N = 32768
L = 16
ROWS = N // L
NC = 2
NS = 16
R_READ = ROWS // NS
R_WRITE = ROWS // (NC * NS)
NEG_INF = float("-inf")

_mesh = plsc.VectorSubcoreMesh(core_axis_name="c", subcore_axis_name="s")


@functools.partial(
    pl.kernel,
    mesh=_mesh,
    compiler_params=pltpu.CompilerParams(needs_layout_passes=False),
    out_type=jax.ShapeDtypeStruct((ROWS, L), jnp.float32),
    scratch_types=[
        pltpu.VMEM((R_READ, L), jnp.float32),
        pltpu.VMEM((R_WRITE, L), jnp.float32),
        pltpu.VMEM((2, L), jnp.float32),
        pltpu.VMEM((L,), jnp.float32),
        pltpu.VMEM((L,), jnp.int32),
        pltpu.VMEM((2 * NS, L), jnp.float32),
        pltpu.VMEM_SHARED((2 * NS, L), jnp.float32),
    ],
)
def _argmax_onehot(x_hbm, out_hbm, xv, ov, stage, tm, ti, lms, sh):
    cid = lax.axis_index("c")
    sid = lax.axis_index("s")

    rrow0 = sid * R_READ
    pltpu.sync_copy(x_hbm.at[pl.ds(rrow0, R_READ), :], xv)

    lane = lax.broadcasted_iota(jnp.int32, (L,), 0)

    def rbody(j, carry):
        mv, iv, cur = carry
        v = xv[j]
        better = v > mv
        mv = jnp.where(better, v, mv)
        iv = jnp.where(better, cur, iv)
        return mv, iv, cur + L

    mv0 = jnp.full((L,), NEG_INF, jnp.float32)
    iv0 = jnp.zeros((L,), jnp.int32)
    cur0 = lane + rrow0 * L
    mv, iv, _ = lax.fori_loop(0, R_READ, rbody, (mv0, iv0, cur0))

    stage[0] = mv
    stage[1] = plsc.bitcast(iv, jnp.float32)
    pltpu.sync_copy(stage, sh.at[pl.ds(2 * sid, 2)])
    plsc.subcore_barrier()

    pltpu.sync_copy(sh, lms)

    def mbody(t, carry):
        gm, gi = carry
        rm = lms[2 * t]
        ri = plsc.bitcast(lms[2 * t + 1], jnp.int32)
        better = rm > gm
        gm = jnp.where(better, rm, gm)
        gi = jnp.where(better, ri, gi)
        return gm, gi

    gm0 = jnp.full((L,), NEG_INF, jnp.float32)
    gi0 = jnp.zeros((L,), jnp.int32)
    gm, gi = lax.fori_loop(0, NS, mbody, (gm0, gi0))

    for s in (8, 4, 2, 1):
        perm = lane ^ s
        tm[...] = gm
        ti[...] = gi
        om = plsc.load_gather(tm, [perm])
        oi = plsc.load_gather(ti, [perm])
        take = (om > gm) | ((om == gm) & (oi < gi))
        gm = jnp.where(take, om, gm)
        gi = jnp.where(take, oi, gi)

    wid = sid * NC + cid
    wrow0 = wid * R_WRITE

    def wbody(j, cur):
        ov[j] = jnp.where(cur == gi, 1.0, 0.0).astype(jnp.float32)
        return cur + L

    lax.fori_loop(0, R_WRITE, wbody, lane + wrow0 * L)
    pltpu.sync_copy(ov, out_hbm.at[pl.ds(wrow0, R_WRITE), :])


def kernel(x):
    out2d = _argmax_onehot(x.reshape(ROWS, L))
    return out2d.reshape(N)


# speedup vs baseline: 1.0428x; 1.0428x over previous
import functools

import jax
import jax.numpy as jnp
from jax import lax
from jax.experimental import pallas as pl
from jax.experimental.pallas import tpu as pltpu
from jax.experimental.pallas import tpu_sc as plsc

N = 32768
L = 16
ROWS = N // L
NS = 16
R_TILE = ROWS // NS
R_UNROLL = 8
W_UNROLL = 8
NEG_INF = float("-inf")

_mesh = plsc.VectorSubcoreMesh(
    core_axis_name="c", subcore_axis_name="s", num_cores=1
)


@functools.partial(
    pl.kernel,
    mesh=_mesh,
    compiler_params=pltpu.CompilerParams(needs_layout_passes=False),
    out_type=jax.ShapeDtypeStruct((ROWS, L), jnp.float32),
    scratch_types=[
        pltpu.VMEM((R_TILE, L), jnp.float32),
        pltpu.VMEM((R_TILE, L), jnp.float32),
        pltpu.VMEM((2, L), jnp.float32),
        pltpu.VMEM((L,), jnp.float32),
        pltpu.VMEM((L,), jnp.int32),
        pltpu.VMEM((2 * NS, L), jnp.float32),
        pltpu.VMEM_SHARED((2 * NS, L), jnp.float32),
    ],
)
def _argmax_onehot(x_hbm, out_hbm, xv, ov, stage, tm, ti, lms, sh):
    sid = lax.axis_index("s")

    row0 = sid * R_TILE
    pltpu.sync_copy(x_hbm.at[pl.ds(row0, R_TILE), :], xv)

    lane = lax.broadcasted_iota(jnp.int32, (L,), 0)

    def rbody(j, carry):
        mv, iv, cur = carry
        base = j * R_UNROLL
        for k in range(R_UNROLL):
            v = xv[base + k]
            idx = cur + k * L
            better = v > mv
            mv = jnp.where(better, v, mv)
            iv = jnp.where(better, idx, iv)
        return mv, iv, cur + R_UNROLL * L

    mv0 = jnp.full((L,), NEG_INF, jnp.float32)
    iv0 = jnp.zeros((L,), jnp.int32)
    cur0 = lane + row0 * L
    mv, iv, _ = lax.fori_loop(0, R_TILE // R_UNROLL, rbody, (mv0, iv0, cur0))

    stage[0] = mv
    stage[1] = plsc.bitcast(iv, jnp.float32)
    pltpu.sync_copy(stage, sh.at[pl.ds(2 * sid, 2)])
    plsc.subcore_barrier()

    pltpu.sync_copy(sh, lms)

    gm = jnp.full((L,), NEG_INF, jnp.float32)
    gi = jnp.zeros((L,), jnp.int32)
    for t in range(NS):
        rm = lms[2 * t]
        ri = plsc.bitcast(lms[2 * t + 1], jnp.int32)
        better = rm > gm
        gm = jnp.where(better, rm, gm)
        gi = jnp.where(better, ri, gi)

    for s in (8, 4, 2, 1):
        perm = lane ^ s
        tm[...] = gm
        ti[...] = gi
        om = plsc.load_gather(tm, [perm])
        oi = plsc.load_gather(ti, [perm])
        take = (om > gm) | ((om == gm) & (oi < gi))
        gm = jnp.where(take, om, gm)
        gi = jnp.where(take, oi, gi)

    def wbody(j, cur):
        base = j * W_UNROLL
        for k in range(W_UNROLL):
            ov[base + k] = jnp.where(cur + k * L == gi, 1.0, 0.0).astype(
                jnp.float32
            )
        return cur + W_UNROLL * L

    lax.fori_loop(0, R_TILE // W_UNROLL, wbody, lane + row0 * L)
    pltpu.sync_copy(ov, out_hbm.at[pl.ds(row0, R_TILE), :])


def kernel(x):
    out2d = _argmax_onehot(x.reshape(ROWS, L))
    return out2d.reshape(N)


# speedup vs baseline: 1.1877x; 1.1389x over previous
import functools

import jax
import jax.numpy as jnp
from jax import lax
from jax.experimental import pallas as pl
from jax.experimental.pallas import tpu as pltpu
from jax.experimental.pallas import tpu_sc as plsc

N = 32768
L = 16
ROWS = N // L
WIDE = 128
WROWS = N // WIDE
NS = 16
R_TILE = ROWS // NS
W_TILE = WROWS // NS
R_UNROLL = 8
NEG_INF = float("-inf")

_mesh = plsc.VectorSubcoreMesh(
    core_axis_name="c", subcore_axis_name="s", num_cores=1
)


@functools.partial(
    pl.kernel,
    mesh=_mesh,
    compiler_params=pltpu.CompilerParams(needs_layout_passes=False),
    out_type=jax.ShapeDtypeStruct((WROWS, WIDE), jnp.float32),
    scratch_types=[
        pltpu.VMEM((R_TILE, L), jnp.float32),
        pltpu.VMEM((W_TILE, WIDE), jnp.float32),
        pltpu.VMEM((2, L), jnp.float32),
        pltpu.VMEM((L,), jnp.float32),
        pltpu.VMEM((L,), jnp.int32),
        pltpu.VMEM((2 * NS, L), jnp.float32),
        pltpu.VMEM_SHARED((2 * NS, L), jnp.float32),
        pltpu.SemaphoreType.DMA,
        pltpu.SemaphoreType.DMA,
    ],
)
def _argmax_onehot(x_hbm, out_hbm, xv, ov, stage, tm, ti, lms, sh,
                   sem_in, sem_out):
    sid = lax.axis_index("s")
    row0 = sid * R_TILE
    wrow0 = sid * W_TILE

    in_dma = pltpu.async_copy(x_hbm.at[pl.ds(row0, R_TILE), :], xv, sem_in)

    zero = jnp.zeros((L,), jnp.float32)
    for r in range(W_TILE):
        for c in range(WIDE // L):
            ov[r, pl.ds(c * L, L)] = zero
    out_dma = pltpu.async_copy(ov, out_hbm.at[pl.ds(wrow0, W_TILE), :], sem_out)

    in_dma.wait()
    lane = lax.broadcasted_iota(jnp.int32, (L,), 0)

    def rbody(j, carry):
        mv, iv, cur = carry
        base = j * R_UNROLL
        for k in range(R_UNROLL):
            v = xv[base + k]
            idx = cur + k * L
            better = v > mv
            mv = jnp.where(better, v, mv)
            iv = jnp.where(better, idx, iv)
        return mv, iv, cur + R_UNROLL * L

    mv0 = jnp.full((L,), NEG_INF, jnp.float32)
    iv0 = jnp.zeros((L,), jnp.int32)
    cur0 = lane + row0 * L
    mv, iv, _ = lax.fori_loop(0, R_TILE // R_UNROLL, rbody, (mv0, iv0, cur0))

    stage[0] = mv
    stage[1] = plsc.bitcast(iv, jnp.float32)
    pltpu.sync_copy(stage, sh.at[pl.ds(2 * sid, 2)])
    plsc.subcore_barrier()

    pltpu.sync_copy(sh, lms)

    gm = jnp.full((L,), NEG_INF, jnp.float32)
    gi = jnp.zeros((L,), jnp.int32)
    for t in range(NS):
        rm = lms[2 * t]
        ri = plsc.bitcast(lms[2 * t + 1], jnp.int32)
        better = rm > gm
        gm = jnp.where(better, rm, gm)
        gi = jnp.where(better, ri, gi)

    for s in (8, 4, 2, 1):
        perm = lane ^ s
        tm[...] = gm
        ti[...] = gi
        om = plsc.load_gather(tm, [perm])
        oi = plsc.load_gather(ti, [perm])
        take = (om > gm) | ((om == gm) & (oi < gi))
        gm = jnp.where(take, om, gm)
        gi = jnp.where(take, oi, gi)

    out_dma.wait()
    wr = (gi >> 7)[0]
    owner = (wr >= wrow0) & (wr < wrow0 + W_TILE)

    @pl.when(owner)
    def _():
        off = gi & (WIDE - 1)
        rloc = wr - wrow0
        for c in range(WIDE // L):
            ov[rloc, pl.ds(c * L, L)] = jnp.where(
                lane + c * L == off, 1.0, 0.0
            ).astype(jnp.float32)
        pltpu.sync_copy(ov, out_hbm.at[pl.ds(wrow0, W_TILE), :])


def kernel(x):
    out2d = _argmax_onehot(x.reshape(ROWS, L))
    return out2d.reshape(N)
